# baseline (speedup 1.0000x reference)
import functools

import jax
import jax.numpy as jnp
from jax import lax
from jax.experimental import pallas as pl
from jax.experimental.pallas import tpu as pltpu
from jax.experimental.pallas import tpu_sc as plsc

NUM_ENTITIES = 1000000
NUM_RELATIONS = 1000
EMBED_DIM = 64
BATCH = 16384

NC = 2
NS = 16
L = 16
NW = NC * NS
B_PER_W = BATCH // NW
CHUNK = 128
N_CHUNKS = B_PER_W // CHUNK

_mesh = plsc.VectorSubcoreMesh(
    core_axis_name="c", subcore_axis_name="s", num_cores=NC, num_subcores=NS
)


@functools.partial(
    pl.kernel,
    out_type=jax.ShapeDtypeStruct((BATCH,), jnp.float32),
    mesh=_mesh,
    compiler_params=pltpu.CompilerParams(use_tc_tiling_on_sc=False),
    scratch_types=[
        pltpu.VMEM((N_CHUNKS, CHUNK), jnp.int32),
        pltpu.VMEM((N_CHUNKS, CHUNK), jnp.int32),
        pltpu.VMEM((N_CHUNKS, CHUNK), jnp.int32),
        pltpu.VMEM((B_PER_W, EMBED_DIM), jnp.float32),
        pltpu.VMEM((B_PER_W, EMBED_DIM), jnp.float32),
        pltpu.VMEM((B_PER_W, EMBED_DIM), jnp.float32),
        pltpu.VMEM((B_PER_W,), jnp.float32),
        pltpu.SemaphoreType.DMA,
    ],
)
def _distmult_sc(h_hbm, r_hbm, t_hbm, ent_hbm, rel_hbm, out_hbm,
                 h_idx, r_idx, t_idx, h_rows, r_rows, t_rows, out_v, sem):
    wid = lax.axis_index("s") * NC + lax.axis_index("c")
    base = wid * B_PER_W

    pltpu.sync_copy(h_hbm.at[wid], h_idx)
    pltpu.sync_copy(r_hbm.at[wid], r_idx)
    pltpu.sync_copy(t_hbm.at[wid], t_idx)

    copies = []
    for c in range(N_CHUNKS):
        rows = pl.ds(c * CHUNK, CHUNK)
        copies.append(pltpu.async_copy(ent_hbm.at[h_idx.at[c]], h_rows.at[rows], sem))
        copies.append(pltpu.async_copy(ent_hbm.at[t_idx.at[c]], t_rows.at[rows], sem))
        copies.append(pltpu.async_copy(rel_hbm.at[r_idx.at[c]], r_rows.at[rows], sem))
    for cp in copies:
        cp.wait()

    lane = lax.iota(jnp.int32, L)
    perms = [lane ^ (1 << s) for s in range(4)]
    masks = [(lane & (1 << s)) == 0 for s in range(4)]

    def group_body(g, _):
        row0 = g * L
        vs = []
        for j in range(L):
            row = row0 + j
            acc = None
            for c in range(EMBED_DIM // L):
                cols = pl.ds(c * L, L)
                p = h_rows[row, cols] * r_rows[row, cols] * t_rows[row, cols]
                acc = p if acc is None else acc + p
            vs.append(acc)
        for s in range(4):
            nxt = []
            for i in range(0, len(vs), 2):
                a, b = vs[i], vs[i + 1]
                a_sw = a[perms[s]]
                b_sw = b[perms[s]]
                u = jnp.where(masks[s], a, b_sw)
                v = jnp.where(masks[s], a_sw, b)
                nxt.append(u + v)
            vs = nxt
        out_v[pl.ds(row0, L)] = vs[0]
        return 0

    lax.fori_loop(0, B_PER_W // L, group_body, 0)

    pltpu.sync_copy(out_v, out_hbm.at[pl.ds(base, B_PER_W)])


def kernel(h, r, t, entity_emb, rel_emb):
    h2 = h.astype(jnp.int32).reshape(NW, N_CHUNKS, CHUNK)
    r2 = r.astype(jnp.int32).reshape(NW, N_CHUNKS, CHUNK)
    t2 = t.astype(jnp.int32).reshape(NW, N_CHUNKS, CHUNK)
    return _distmult_sc(h2, r2, t2, entity_emb, rel_emb)
